# Initial kernel scaffold; baseline (speedup 1.0000x reference)
import jax
import jax.numpy as jnp
from jax.experimental import pallas as pl


def kernel(x, enc_w1, enc_b1, enc_w2, enc_b2, enc_w3, enc_b3, dec_w1, dec_b1, dec_w2, dec_b2, dec_w3, dec_b3, codebook):
    raise NotImplementedError("write your pallas kernel here")


# speedup vs baseline: 1.1862x; 1.1862x over previous
import functools

import jax
import jax.numpy as jnp
from jax import lax
from jax.experimental import pallas as pl
from jax.experimental.pallas import tpu as pltpu
from jax.experimental.pallas import tpu_sc as plsc

_B, _D, _L, _K = 9216, 768, 256, 8192
_BLK = 256
_NBLK = _B // _BLK

_NW = 32
_BPW = _B // _NW
_CH = 96
_NCH = _BPW // _CH


def _encode_vq_body(x_ref, w1_ref, b1_ref, w2_ref, b2_ref, w3_ref, b3_ref,
                    cb_ref, ze_ref, idx_ref, csum_ref):
    i = pl.program_id(0)

    @pl.when(i == 0)
    def _():
        cb = cb_ref[...]
        csum_ref[...] = jnp.sum(cb * cb, axis=1)[None, :]

    x = x_ref[...]
    h = jnp.maximum(jnp.dot(x, w1_ref[...]) + b1_ref[...], 0.0)
    h = jnp.maximum(jnp.dot(h, w2_ref[...]) + b2_ref[...], 0.0)
    z = jnp.dot(h, w3_ref[...]) + b3_ref[...]
    ze_ref[...] = z

    ab = lax.dot_general(z, cb_ref[...], (((1,), (1,)), ((), ())))
    rowsum = jnp.sum(z * z, axis=1, keepdims=True)
    d2 = (rowsum - 2.0 * ab) + csum_ref[...]
    m = jnp.min(d2, axis=1, keepdims=True)
    col = lax.broadcasted_iota(jnp.int32, (_BLK, _K), 1)
    idx = jnp.min(jnp.where(d2 == m, col, _K), axis=1)
    idx_ref[...] = idx[None, None, :]


def _encode_vq(x, w1, b1, w2, b2, w3, b3, cb):
    return pl.pallas_call(
        _encode_vq_body,
        grid=(_NBLK,),
        in_specs=[
            pl.BlockSpec((_BLK, _D), lambda i: (i, 0)),
            pl.BlockSpec((_D, 256), lambda i: (0, 0)),
            pl.BlockSpec((256,), lambda i: (0,)),
            pl.BlockSpec((256, 256), lambda i: (0, 0)),
            pl.BlockSpec((256,), lambda i: (0,)),
            pl.BlockSpec((256, _L), lambda i: (0, 0)),
            pl.BlockSpec((_L,), lambda i: (0,)),
            pl.BlockSpec((_K, _L), lambda i: (0, 0)),
        ],
        out_specs=[
            pl.BlockSpec((_BLK, _L), lambda i: (i, 0)),
            pl.BlockSpec((1, 1, _BLK), lambda i: (i, 0, 0)),
        ],
        out_shape=[
            jax.ShapeDtypeStruct((_B, _L), jnp.float32),
            jax.ShapeDtypeStruct((_NBLK, 1, _BLK), jnp.int32),
        ],
        scratch_shapes=[pltpu.VMEM((1, _K), jnp.float32)],
    )(x, w1, b1, w2, b2, w3, b3, cb)


def _sc_gather(codebook, idx3):
    mesh = plsc.VectorSubcoreMesh(core_axis_name="c", subcore_axis_name="s")

    @functools.partial(
        pl.kernel,
        out_type=jax.ShapeDtypeStruct((_B, _L), jnp.float32),
        mesh=mesh,
        scratch_types=[
            pltpu.VMEM((_NCH, _CH), jnp.int32),
            pltpu.VMEM((_BPW, _L), jnp.float32),
            pltpu.SemaphoreType.DMA,
        ],
    )
    def k(cb_hbm, idx_hbm, out_hbm, idx_v, rows_v, sem):
        wid = lax.axis_index("s") * 2 + lax.axis_index("c")
        base = wid * _BPW
        pltpu.sync_copy(idx_hbm.at[wid], idx_v)
        copies = [
            pltpu.async_copy(
                cb_hbm.at[idx_v.at[j]],
                rows_v.at[pl.ds(j * _CH, _CH)],
                sem,
            )
            for j in range(_NCH)
        ]
        for c in copies:
            c.wait()
        pltpu.sync_copy(rows_v, out_hbm.at[pl.ds(base, _BPW)])

    return k(codebook, idx3)


def _decode_body(ze_ref, zq_ref, w1_ref, b1_ref, w2_ref, b2_ref, w3_ref,
                 b3_ref, xrec_ref, loss_ref, acc_ref):
    i = pl.program_id(0)
    ze = ze_ref[...]
    zq = zq_ref[...]
    zst = ze + (zq - ze)
    h = jnp.maximum(jnp.dot(zst, w1_ref[...]) + b1_ref[...], 0.0)
    h = jnp.maximum(jnp.dot(h, w2_ref[...]) + b2_ref[...], 0.0)
    xrec_ref[...] = jnp.dot(h, w3_ref[...]) + b3_ref[...]

    diff = ze - zq
    part = jnp.sum(diff * diff)

    @pl.when(i == 0)
    def _():
        acc_ref[0] = 0.0

    acc_ref[0] += part

    @pl.when(i == _NBLK - 1)
    def _():
        loss_ref[...] = (acc_ref[0] / jnp.float32(_B * _L)).reshape(1, 1)


def _decode(ze, zq, w1, b1, w2, b2, w3, b3):
    return pl.pallas_call(
        _decode_body,
        grid=(_NBLK,),
        in_specs=[
            pl.BlockSpec((_BLK, _L), lambda i: (i, 0)),
            pl.BlockSpec((_BLK, _L), lambda i: (i, 0)),
            pl.BlockSpec((_L, 256), lambda i: (0, 0)),
            pl.BlockSpec((256,), lambda i: (0,)),
            pl.BlockSpec((256, 256), lambda i: (0, 0)),
            pl.BlockSpec((256,), lambda i: (0,)),
            pl.BlockSpec((256, _D), lambda i: (0, 0)),
            pl.BlockSpec((_D,), lambda i: (0,)),
        ],
        out_specs=[
            pl.BlockSpec((_BLK, _D), lambda i: (i, 0)),
            pl.BlockSpec((1, 1), lambda i: (0, 0)),
        ],
        out_shape=[
            jax.ShapeDtypeStruct((_B, _D), jnp.float32),
            jax.ShapeDtypeStruct((1, 1), jnp.float32),
        ],
        scratch_shapes=[pltpu.SMEM((1,), jnp.float32)],
    )(ze, zq, w1, b1, w2, b2, w3, b3)


def kernel(x, enc_w1, enc_b1, enc_w2, enc_b2, enc_w3, enc_b3,
           dec_w1, dec_b1, dec_w2, dec_b2, dec_w3, dec_b3, codebook):
    ze, idx3 = _encode_vq(x, enc_w1, enc_b1, enc_w2, enc_b2, enc_w3, enc_b3,
                          codebook)
    idx_flat = idx3.reshape(_B)
    zq = _sc_gather(codebook, idx_flat.reshape(_NW, _NCH, _CH))
    xrec, loss = _decode(ze, zq, dec_w1, dec_b1, dec_w2, dec_b2,
                         dec_w3, dec_b3)
    return xrec, zq, jnp.reshape(loss, ()), idx_flat
